# Initial kernel scaffold; baseline (speedup 1.0000x reference)
import jax
import jax.numpy as jnp
from jax import lax
from jax.experimental import pallas as pl
from jax.experimental.pallas import tpu as pltpu
from jax.experimental.pallas import tpu_sc as plsc

_EPS = 1e-08
N = 10000
E = 320000
D = 128

NC = 2
NS = 16
NW = NC * NS
E_PER_TILE = E // NW
K = 80
NCHUNK = E_PER_TILE // K
ZROWS = 125
ACC_ROWS = N // NS
HIST_PAD = 10240
HIST_PER_TILE = HIST_PAD // NS

BLK = 400


def _sc_scatter(row, col, h):
  mesh = plsc.VectorSubcoreMesh(core_axis_name="c", subcore_axis_name="s")

  def body(row_hbm, col_hbm, h_hbm, s_hbm, cnt_hbm,
           idx_col, idx_row, rows_v, ones_v, zbuf, zhist, acc_sh, hist_sh,
           sem):
    cid = lax.axis_index("c")
    sid = lax.axis_index("s")
    wid = sid * NC + cid

    zeros16 = jnp.zeros((16,), jnp.float32)
    ones16 = jnp.ones((16,), jnp.float32)

    @pl.loop(0, ZROWS)
    def _zero_zbuf(i):
      for j in range(D // 16):
        zbuf[i, pl.ds(j * 16, 16)] = zeros16

    for j in range(HIST_PER_TILE // 16):
      zhist[pl.ds(j * 16, 16)] = zeros16
    for j in range(K // 16):
      ones_v[pl.ds(j * 16, 16)] = ones16

    for r in range(ACC_ROWS // ZROWS):
      pltpu.sync_copy(zbuf,
                      acc_sh.at[pl.ds(sid * ACC_ROWS + r * ZROWS, ZROWS)])
    pltpu.sync_copy(zhist, hist_sh.at[pl.ds(sid * HIST_PER_TILE,
                                            HIST_PER_TILE)])
    plsc.subcore_barrier()

    base0 = wid * E_PER_TILE

    @pl.loop(0, NCHUNK)
    def _chunk(c):
      base = base0 + c * K
      pltpu.sync_copy(col_hbm.at[pl.ds(base, K)], idx_col)
      pltpu.sync_copy(row_hbm.at[pl.ds(base, K)], idx_row)
      pltpu.async_copy(h_hbm.at[idx_col], rows_v, sem).wait()
      pltpu.sync_copy(rows_v, acc_sh.at[idx_row], add=True)
      pltpu.sync_copy(ones_v, hist_sh.at[idx_row], add=True)

    plsc.subcore_barrier()
    pltpu.sync_copy(acc_sh.at[pl.ds(sid * ACC_ROWS, ACC_ROWS)],
                    s_hbm.at[cid].at[pl.ds(sid * ACC_ROWS, ACC_ROWS)])
    pltpu.sync_copy(hist_sh.at[pl.ds(sid * HIST_PER_TILE, HIST_PER_TILE)],
                    cnt_hbm.at[cid].at[pl.ds(sid * HIST_PER_TILE,
                                             HIST_PER_TILE)])

  run = pl.kernel(
      body,
      out_type=(
          jax.ShapeDtypeStruct((NC, N, D), jnp.float32),
          jax.ShapeDtypeStruct((NC, HIST_PAD), jnp.float32),
      ),
      mesh=mesh,
      scratch_types=[
          pltpu.VMEM((K,), jnp.int32),
          pltpu.VMEM((K,), jnp.int32),
          pltpu.VMEM((K, D), jnp.float32),
          pltpu.VMEM((K,), jnp.float32),
          pltpu.VMEM((ZROWS, D), jnp.float32),
          pltpu.VMEM((HIST_PER_TILE,), jnp.float32),
          pltpu.VMEM_SHARED((N, D), jnp.float32),
          pltpu.VMEM_SHARED((HIST_PAD,), jnp.float32),
          pltpu.SemaphoreType.DMA,
      ],
  )
  return run(row, col, h)


def _tc_h(x, fc_wt, fc_b):
  def body(x_ref, w_ref, b_ref, o_ref):
    o_ref[...] = (jnp.dot(x_ref[...], w_ref[...],
                          preferred_element_type=jnp.float32) + b_ref[...])

  return pl.pallas_call(
      body,
      grid=(N // BLK,),
      in_specs=[
          pl.BlockSpec((BLK, D), lambda i: (i, 0)),
          pl.BlockSpec((D, D), lambda i: (0, 0)),
          pl.BlockSpec((1, D), lambda i: (0, 0)),
      ],
      out_specs=pl.BlockSpec((BLK, D), lambda i: (i, 0)),
      out_shape=jax.ShapeDtypeStruct((N, D), jnp.float32),
  )(x, fc_wt, fc_b)


def _tc_combine(x, h, s0, s1, cnt0, cnt1, degree,
                dir_wt, dir_b, neu_wt, neu_b, rob_wt, rob_b):
  def body(x_ref, h_ref, s0_ref, s1_ref, c0_ref, c1_ref, deg_ref,
           dw_ref, db_ref, nw_ref, nb_ref, rw_ref, rb_ref, o_ref):
    xb = x_ref[...]
    alpha = jnp.maximum(
        jnp.dot(xb, dw_ref[...], preferred_element_type=jnp.float32)
        + db_ref[...], 0.0)
    beta = jnp.maximum(
        jnp.dot(xb, nw_ref[...], preferred_element_type=jnp.float32)
        + nb_ref[...], 0.0)
    gamma = (jnp.dot(xb, rw_ref[...], preferred_element_type=jnp.float32)
             + rb_ref[...])
    cnt = c0_ref[...] + c1_ref[...]
    agg = cnt * h_ref[...] + s0_ref[...] + s1_ref[...]
    o_ref[...] = (beta * agg + gamma) / (alpha + beta * deg_ref[...] + _EPS)

  row_spec = pl.BlockSpec((BLK, D), lambda i: (i, 0))
  col1_spec = pl.BlockSpec((BLK, 1), lambda i: (i, 0))
  w_spec = pl.BlockSpec((D, D), lambda i: (0, 0))
  b_spec = pl.BlockSpec((1, D), lambda i: (0, 0))
  return pl.pallas_call(
      body,
      grid=(N // BLK,),
      in_specs=[row_spec, row_spec, row_spec, row_spec,
                col1_spec, col1_spec, col1_spec,
                w_spec, b_spec, w_spec, b_spec, w_spec, b_spec],
      out_specs=row_spec,
      out_shape=jax.ShapeDtypeStruct((N, D), jnp.float32),
  )(x, h, s0, s1, cnt0, cnt1, degree,
    dir_wt, dir_b, neu_wt, neu_b, rob_wt, rob_b)


def kernel(x, edge_index, degree, fc_w, fc_b,
           dir_w, dir_b, neu_w, neu_b, rob_w, rob_b):
  ei = edge_index.astype(jnp.int32)
  row = ei[0]
  col = ei[1]
  h = _tc_h(x, fc_w.T, fc_b.reshape(1, D))
  s_part, cnt_part = _sc_scatter(row, col, h)
  s0 = s_part[0]
  s1 = s_part[1]
  cnt0 = cnt_part[0, :N].reshape(N, 1)
  cnt1 = cnt_part[1, :N].reshape(N, 1)
  return _tc_combine(x, h, s0, s1, cnt0, cnt1, degree,
                     dir_w.T, dir_b.reshape(1, D),
                     neu_w.T, neu_b.reshape(1, D),
                     rob_w.T, rob_b.reshape(1, D))


# speedup vs baseline: 6.4452x; 6.4452x over previous
import jax
import jax.numpy as jnp
from jax import lax
from jax.experimental import pallas as pl
from jax.experimental.pallas import tpu as pltpu
from jax.experimental.pallas import tpu_sc as plsc

_EPS = 1e-08
N = 10000
E = 320000
D = 128

NC = 2
NS = 16
NW = NC * NS
E_PER_TILE = E // NW
K = 80
NCHUNK = E_PER_TILE // K
N_PAD = 10240
ZROWS = 128
ACC_ROWS = N_PAD // NS
HIST_PER_TILE = N_PAD // NS

BLK = 400


def _sc_scatter(row, col, h):
  mesh = plsc.VectorSubcoreMesh(core_axis_name="c", subcore_axis_name="s",
                                num_cores=NC, num_subcores=NS)

  def body(row_hbm, col_hbm, h_hbm, s_hbm, cnt_hbm,
           idx_col, idx_row, rows_v, ones_v, zbuf, zhist, acc_sh, hist_sh,
           sem):
    cid = lax.axis_index("c")
    sid = lax.axis_index("s")
    wid = sid * NC + cid

    zeros16 = jnp.zeros((16,), jnp.float32)
    ones16 = jnp.ones((16,), jnp.float32)

    @pl.loop(0, ZROWS)
    def _zero_zbuf(i):
      for j in range(D // 16):
        zbuf[i, pl.ds(j * 16, 16)] = zeros16

    for j in range(HIST_PER_TILE // 16):
      zhist[pl.ds(j * 16, 16)] = zeros16
    for j in range(K // 16):
      ones_v[pl.ds(j * 16, 16)] = ones16

    for r in range(ACC_ROWS // ZROWS):
      pltpu.sync_copy(zbuf,
                      acc_sh.at[pl.ds(sid * ACC_ROWS + r * ZROWS, ZROWS)])
    pltpu.sync_copy(zhist, hist_sh.at[pl.ds(sid * HIST_PER_TILE,
                                            HIST_PER_TILE)])
    plsc.subcore_barrier()

    base0 = wid * E_PER_TILE

    @pl.loop(0, NCHUNK)
    def _chunk(c):
      base = base0 + c * K
      pltpu.sync_copy(col_hbm.at[pl.ds(base, K)], idx_col)
      pltpu.sync_copy(row_hbm.at[pl.ds(base, K)], idx_row)
      pltpu.async_copy(h_hbm.at[idx_col], rows_v, sem).wait()
      pltpu.sync_copy(rows_v, acc_sh.at[idx_row], add=True)
      pltpu.sync_copy(ones_v, hist_sh.at[idx_row], add=True)

    plsc.subcore_barrier()
    pltpu.sync_copy(acc_sh.at[pl.ds(sid * ACC_ROWS, ACC_ROWS)],
                    s_hbm.at[cid].at[pl.ds(sid * ACC_ROWS, ACC_ROWS)])
    pltpu.sync_copy(hist_sh.at[pl.ds(sid * HIST_PER_TILE, HIST_PER_TILE)],
                    cnt_hbm.at[cid].at[pl.ds(sid * HIST_PER_TILE,
                                             HIST_PER_TILE)])

  run = pl.kernel(
      body,
      out_type=(
          jax.ShapeDtypeStruct((NC, N_PAD, D), jnp.float32),
          jax.ShapeDtypeStruct((NC, N_PAD), jnp.float32),
      ),
      mesh=mesh,
      scratch_types=[
          pltpu.VMEM((K,), jnp.int32),
          pltpu.VMEM((K,), jnp.int32),
          pltpu.VMEM((K, D), jnp.float32),
          pltpu.VMEM((K,), jnp.float32),
          pltpu.VMEM((ZROWS, D), jnp.float32),
          pltpu.VMEM((HIST_PER_TILE,), jnp.float32),
          pltpu.VMEM_SHARED((N_PAD, D), jnp.float32),
          pltpu.VMEM_SHARED((N_PAD,), jnp.float32),
          pltpu.SemaphoreType.DMA,
      ],
  )
  return run(row, col, h)


def _tc_h(x, fc_wt, fc_b):
  def body(x_ref, w_ref, b_ref, o_ref):
    o_ref[...] = (jnp.dot(x_ref[...], w_ref[...],
                          preferred_element_type=jnp.float32) + b_ref[...])

  return pl.pallas_call(
      body,
      grid=(N // BLK,),
      in_specs=[
          pl.BlockSpec((BLK, D), lambda i: (i, 0)),
          pl.BlockSpec((D, D), lambda i: (0, 0)),
          pl.BlockSpec((1, D), lambda i: (0, 0)),
      ],
      out_specs=pl.BlockSpec((BLK, D), lambda i: (i, 0)),
      out_shape=jax.ShapeDtypeStruct((N, D), jnp.float32),
  )(x, fc_wt, fc_b)


def _tc_combine(x, h, s0, s1, cnt0, cnt1, degree,
                dir_wt, dir_b, neu_wt, neu_b, rob_wt, rob_b):
  def body(x_ref, h_ref, s0_ref, s1_ref, c0_ref, c1_ref, deg_ref,
           dw_ref, db_ref, nw_ref, nb_ref, rw_ref, rb_ref, o_ref):
    xb = x_ref[...]
    alpha = jnp.maximum(
        jnp.dot(xb, dw_ref[...], preferred_element_type=jnp.float32)
        + db_ref[...], 0.0)
    beta = jnp.maximum(
        jnp.dot(xb, nw_ref[...], preferred_element_type=jnp.float32)
        + nb_ref[...], 0.0)
    gamma = (jnp.dot(xb, rw_ref[...], preferred_element_type=jnp.float32)
             + rb_ref[...])
    cnt = c0_ref[...] + c1_ref[...]
    agg = cnt * h_ref[...] + s0_ref[...] + s1_ref[...]
    o_ref[...] = (beta * agg + gamma) / (alpha + beta * deg_ref[...] + _EPS)

  row_spec = pl.BlockSpec((BLK, D), lambda i: (i, 0))
  col1_spec = pl.BlockSpec((BLK, 1), lambda i: (i, 0))
  w_spec = pl.BlockSpec((D, D), lambda i: (0, 0))
  b_spec = pl.BlockSpec((1, D), lambda i: (0, 0))
  return pl.pallas_call(
      body,
      grid=(N // BLK,),
      in_specs=[row_spec, row_spec, row_spec, row_spec,
                col1_spec, col1_spec, col1_spec,
                w_spec, b_spec, w_spec, b_spec, w_spec, b_spec],
      out_specs=row_spec,
      out_shape=jax.ShapeDtypeStruct((N, D), jnp.float32),
  )(x, h, s0, s1, cnt0, cnt1, degree,
    dir_wt, dir_b, neu_wt, neu_b, rob_wt, rob_b)


def kernel(x, edge_index, degree, fc_w, fc_b,
           dir_w, dir_b, neu_w, neu_b, rob_w, rob_b):
  ei = edge_index.astype(jnp.int32)
  row = ei[0]
  col = ei[1]
  h = _tc_h(x, fc_w.T, fc_b.reshape(1, D))
  s_part, cnt_part = _sc_scatter(row, col, h)
  s0 = s_part[0, :N]
  s1 = s_part[1, :N]
  cnt0 = cnt_part[0, :N].reshape(N, 1)
  cnt1 = cnt_part[1, :N].reshape(N, 1)
  return _tc_combine(x, h, s0, s1, cnt0, cnt1, degree,
                     dir_w.T, dir_b.reshape(1, D),
                     neu_w.T, neu_b.reshape(1, D),
                     rob_w.T, rob_b.reshape(1, D))
